# baseline (speedup 1.0000x reference)
import functools

import jax
import jax.numpy as jnp
from jax import lax
from jax.experimental import pallas as pl
from jax.experimental.pallas import tpu as pltpu
from jax.experimental.pallas import tpu_sc as plsc

B = 16384
NC, NS, L = 2, 16, 16
NW = NC * NS
BPW = B // NW
CHUNK = 128
NCHUNK = BPW // CHUNK


@functools.partial(
    pl.kernel,
    mesh=plsc.VectorSubcoreMesh(core_axis_name="c", subcore_axis_name="s"),
    out_type=jax.ShapeDtypeStruct((B,), jnp.float32),
    scratch_types=[
        pltpu.VMEM((BPW,), jnp.int32),
        pltpu.VMEM((BPW,), jnp.float32),
        pltpu.SemaphoreType.DMA,
    ],
)
def _gather_sigmoid(idx_hbm, alpha_hbm, out_hbm, idx_v, vals_v, sem):
    wid = lax.axis_index("s") * NC + lax.axis_index("c")
    base = wid * BPW

    pltpu.sync_copy(idx_hbm.at[pl.ds(base, BPW)], idx_v)

    copies = [
        pltpu.async_copy(
            alpha_hbm.at[idx_v.at[pl.ds(j * CHUNK, CHUNK)]],
            vals_v.at[pl.ds(j * CHUNK, CHUNK)],
            sem,
        )
        for j in range(NCHUNK)
    ]
    one = jnp.full((L,), 1.0, dtype=jnp.float32)
    for j in range(NCHUNK):
        copies[j].wait()

        def body(i, _):
            x = vals_v[pl.ds(i * L, L)]
            vals_v[pl.ds(i * L, L)] = one / (one + jnp.exp(-x))
            return 0

        lax.fori_loop(j * (CHUNK // L), (j + 1) * (CHUNK // L), body, 0,
                      unroll=2)

    pltpu.sync_copy(vals_v, out_hbm.at[pl.ds(base, BPW)])


def kernel(idx, alpha):
    return _gather_sigmoid(idx.astype(jnp.int32), alpha)


# speedup vs baseline: 1.0324x; 1.0324x over previous
import functools

import jax
import jax.numpy as jnp
from jax import lax
from jax.experimental import pallas as pl
from jax.experimental.pallas import tpu as pltpu
from jax.experimental.pallas import tpu_sc as plsc

B = 16384
NC, NS, L = 2, 16, 16
NW = NC * NS
BPW = B // NW
CHUNK = 128
NCHUNK = BPW // CHUNK


@functools.partial(
    pl.kernel,
    mesh=plsc.VectorSubcoreMesh(core_axis_name="c", subcore_axis_name="s"),
    out_type=jax.ShapeDtypeStruct((B,), jnp.float32),
    scratch_types=[
        pltpu.VMEM((BPW,), jnp.int32),
        pltpu.VMEM((BPW,), jnp.float32),
        pltpu.SemaphoreType.DMA,
    ],
)
def _gather_sigmoid(idx_hbm, alpha_hbm, out_hbm, idx_v, vals_v, sem):
    wid = lax.axis_index("s") * NC + lax.axis_index("c")
    base = wid * BPW

    pltpu.sync_copy(idx_hbm.at[pl.ds(base, BPW)], idx_v)

    pltpu.async_copy(alpha_hbm.at[idx_v], vals_v, sem).wait()

    one = jnp.full((L,), 1.0, dtype=jnp.float32)
    for i in range(BPW // L):
        x = vals_v[pl.ds(i * L, L)]
        vals_v[pl.ds(i * L, L)] = one / (one + jnp.exp(-x))

    pltpu.sync_copy(vals_v, out_hbm.at[pl.ds(base, BPW)])


def kernel(idx, alpha):
    return _gather_sigmoid(idx.astype(jnp.int32), alpha)
